# baseline (speedup 1.0000x reference)
import functools

import jax
import jax.numpy as jnp
from jax import lax
from jax.experimental import pallas as pl
from jax.experimental.pallas import tpu as pltpu
from jax.experimental.pallas import tpu_sc as plsc

NC = 2
NS = 16
NW = NC * NS
NB = 4
NBUF = 4


@functools.partial(jax.jit, static_argnames=("B", "F", "D"))
def _emb_lookup(idx, W, B, F, D):
    b_per_w = B // NW
    n_steps = b_per_w // NB
    assert n_steps % NBUF == 0 and n_steps >= 2 * NBUF

    mesh = plsc.VectorSubcoreMesh(
        core_axis_name="c", subcore_axis_name="s",
        num_cores=NC, num_subcores=NS,
    )

    @functools.partial(
        pl.kernel,
        out_type=jax.ShapeDtypeStruct((B, F, D), jnp.float32),
        mesh=mesh,
        scratch_types=[
            pltpu.VMEM((b_per_w, F), jnp.int32),
            [pltpu.VMEM((NB, F, D), jnp.float32)] * NBUF,
            [pltpu.SemaphoreType.DMA] * NBUF,
            [pltpu.SemaphoreType.DMA] * NBUF,
        ],
        compiler_params=pltpu.CompilerParams(use_tc_tiling_on_sc=False),
    )
    def k(idx_hbm, table_hbm, out_hbm, idx_v, bufs, gsems, ssems):
        cid = lax.axis_index("c")
        sid = lax.axis_index("s")
        wid = sid * NC + cid
        b0 = wid * b_per_w

        def issue_gather(step, b):
            for j in range(NB):
                pltpu.async_copy(
                    table_hbm.at[idx_v.at[step * NB + j]],
                    bufs[b].at[j],
                    gsems[b],
                )

        def wait_gather(b):
            pltpu.make_async_copy(
                out_hbm.at[pl.ds(b0, NB)], bufs[b], gsems[b]
            ).wait()

        def issue_scatter(step, b):
            pltpu.async_copy(
                bufs[b], out_hbm.at[pl.ds(b0 + step * NB, NB)], ssems[b]
            )

        def wait_scatter(b):
            pltpu.make_async_copy(
                bufs[b], out_hbm.at[pl.ds(b0, NB)], ssems[b]
            ).wait()

        pltpu.sync_copy(idx_hbm.at[pl.ds(b0, b_per_w)], idx_v)

        for b in range(NBUF):
            issue_gather(b, b)

        @pl.loop(0, n_steps, step=NBUF)
        def _(o):
            for b in range(NBUF):
                s = o + b
                wait_gather(b)
                issue_scatter(s, b)

                @pl.when(s + NBUF < n_steps)
                def _():
                    wait_scatter(b)
                    issue_gather(s + NBUF, b)

        for b in range(NBUF):
            wait_scatter(b)

    return k(idx, W)


def kernel(input, W):
    B, F = input.shape
    D = W.shape[1]
    return _emb_lookup(input, W, B, F, D)
